# baseline (speedup 1.0000x reference)
import jax
import jax.numpy as jnp
from jax.experimental import pallas as pl
from jax.experimental.pallas import tpu as pltpu

_CLASS_NUM = 80
_CCHUNK = 8
_LEVELS = [64, 32, 16, 8, 4]
_HH = [h * h for h in _LEVELS]
_N = sum(_HH)
_B = 16
_BB = 4
_STEPS = _B // _BB


def _loss_body(cls0, cls1, cls2, cls3, cls4,
               cnt0, cnt1, cnt2, cnt3, cnt4,
               reg0, reg1, reg2, reg3, reg4,
               cls_t, cnt_t, reg_t,
               cls_out, cnt_out, reg_out, npos_out):
    step = pl.program_id(0)

    @pl.when(step == 0)
    def _init():
        cls_out[0, 0] = 0.0
        cnt_out[0, 0] = 0.0
        reg_out[0, 0] = 0.0
        npos_out[0, 0] = 0.0

    cls_sum = 0.0
    cnt_sum = 0.0
    reg_sum = 0.0
    msk_sum = 0.0
    facc = jnp.zeros((_CLASS_NUM, 128), jnp.float32)
    for i in range(_BB):
        off = 0
        for lvl, (cls_ref, cnt_ref, reg_ref, hh) in enumerate(zip(
                (cls0, cls1, cls2, cls3, cls4),
                (cnt0, cnt1, cnt2, cnt3, cnt4),
                (reg0, reg1, reg2, reg3, reg4), _HH)):
            tm1 = cls_t[i][:, off:off + hh] - 1
            tc = cnt_t[i][:, off:off + hh]
            mask = (tc > -1.0).astype(jnp.float32)

            ciota = jax.lax.broadcasted_iota(
                jnp.int32, (_CLASS_NUM, min(hh, 128)), 0)
            for l0 in range(0, hh, 128):
                lw = min(128, hh)
                tm1c = tm1[:, l0:l0 + lw]
                maskc = mask[:, l0:l0 + lw]
                x = cls_ref[i, :, l0:l0 + lw]
                is_t = ciota == tm1c
                p = jax.nn.sigmoid(x)
                pt = jnp.where(is_t, p, 1.0 - p)
                w = jnp.where(is_t, 0.25, 0.75)
                fl = -w * jnp.square(1.0 - pt) * jnp.log(pt + 1e-12)
                if lw == 128:
                    facc = facc + fl * maskc
                else:
                    cls_sum += jnp.sum(fl * maskc)

            xc = cnt_ref[i, :, :]
            bce = (jnp.maximum(xc, 0.0) - xc * tc
                   + jnp.log1p(jnp.exp(-jnp.abs(xc))))
            cnt_sum += jnp.sum(bce * mask)

            rp = reg_ref[i]
            rt = reg_t[i][:, off:off + hh]
            reg_sum += jnp.sum(jnp.abs(rp - rt) * mask)

            msk_sum += jnp.sum(mask)
            off += hh

    cls_sum += jnp.sum(facc)
    cls_out[0, 0] += cls_sum
    cnt_out[0, 0] += cnt_sum
    reg_out[0, 0] += reg_sum
    npos_out[0, 0] += msk_sum


@jax.jit
def kernel(cls_logits_0, cls_logits_1, cls_logits_2, cls_logits_3, cls_logits_4,
           cnt_logits_0, cnt_logits_1, cnt_logits_2, cnt_logits_3, cnt_logits_4,
           reg_preds_0, reg_preds_1, reg_preds_2, reg_preds_3, reg_preds_4,
           cls_targets, cnt_targets, reg_targets):
    cls = [c.reshape(_B, _CLASS_NUM, hh)
           for c, hh in zip((cls_logits_0, cls_logits_1, cls_logits_2,
                             cls_logits_3, cls_logits_4), _HH)]
    cnt = [c.reshape(_B, 1, hh)
           for c, hh in zip((cnt_logits_0, cnt_logits_1, cnt_logits_2,
                             cnt_logits_3, cnt_logits_4), _HH)]
    reg = [r.reshape(_B, 4, hh)
           for r, hh in zip((reg_preds_0, reg_preds_1, reg_preds_2,
                             reg_preds_3, reg_preds_4), _HH)]
    cls_t = cls_targets.reshape(_B, 1, _N)
    cnt_t = cnt_targets.reshape(_B, 1, _N)
    reg_t = jnp.transpose(reg_targets, (0, 2, 1))

    in_specs = (
        [pl.BlockSpec((_BB, _CLASS_NUM, hh), lambda b: (b, 0, 0)) for hh in _HH]
        + [pl.BlockSpec((_BB, 1, hh), lambda b: (b, 0, 0)) for hh in _HH]
        + [pl.BlockSpec((_BB, 4, hh), lambda b: (b, 0, 0)) for hh in _HH]
        + [pl.BlockSpec((_BB, 1, _N), lambda b: (b, 0, 0)),
           pl.BlockSpec((_BB, 1, _N), lambda b: (b, 0, 0)),
           pl.BlockSpec((_BB, 4, _N), lambda b: (b, 0, 0))]
    )
    out_specs = [pl.BlockSpec(memory_space=pltpu.SMEM)] * 4
    out_shape = [jax.ShapeDtypeStruct((1, 1), jnp.float32)] * 4

    cls_s, cnt_s, reg_s, npos = pl.pallas_call(
        _loss_body,
        grid=(_STEPS,),
        in_specs=in_specs,
        out_specs=out_specs,
        out_shape=out_shape,
    )(*cls, *cnt, *reg, cls_t, cnt_t, reg_t)

    num_pos = jnp.maximum(npos[0, 0], 1.0)
    cls_loss = cls_s[0, 0] / num_pos
    cnt_loss = cnt_s[0, 0] / num_pos
    reg_loss = reg_s[0, 0] / num_pos
    return (cls_loss, cnt_loss, reg_loss, cls_loss + cnt_loss + reg_loss)


# speedup vs baseline: 1.0036x; 1.0036x over previous
import jax
import jax.numpy as jnp
from jax.experimental import pallas as pl
from jax.experimental.pallas import tpu as pltpu

_CLASS_NUM = 80
_LEVELS = [64, 32, 16, 8, 4]
_HH = [h * h for h in _LEVELS]
_N = sum(_HH)
_B = 16
_BB = 2
_STEPS = _B // _BB


def _loss_body(cls0, cls1, cls2, cls3, cls4,
               cnt0, cnt1, cnt2, cnt3, cnt4,
               reg0, reg1, reg2, reg3, reg4,
               cls_t, cnt_t, reg_t,
               cls_out, cnt_out, reg_out, npos_out):
    step = pl.program_id(0)

    @pl.when(step == 0)
    def _init():
        cls_out[0, 0] = 0.0
        cnt_out[0, 0] = 0.0
        reg_out[0, 0] = 0.0
        npos_out[0, 0] = 0.0

    cls_sum = 0.0
    cnt_sum = 0.0
    reg_sum = 0.0
    msk_sum = 0.0
    facc = jnp.zeros((_CLASS_NUM, 128), jnp.float32)
    for i in range(_BB):
        off = 0
        for cls_ref, cnt_ref, reg_ref, hh in zip(
                (cls0, cls1, cls2, cls3, cls4),
                (cnt0, cnt1, cnt2, cnt3, cnt4),
                (reg0, reg1, reg2, reg3, reg4), _HH):
            tm1 = cls_t[i][:, off:off + hh] - 1
            tc = cnt_t[i][:, off:off + hh]
            mask = (tc > -1.0).astype(jnp.float32)

            ciota = jax.lax.broadcasted_iota(
                jnp.int32, (_CLASS_NUM, min(hh, 128)), 0)
            for l0 in range(0, hh, 128):
                lw = min(128, hh)
                tm1c = tm1[:, l0:l0 + lw]
                maskc = mask[:, l0:l0 + lw]
                x = cls_ref[i, :, l0:l0 + lw]
                is_t = ciota == tm1c
                p = jax.nn.sigmoid(x)
                pt = jnp.where(is_t, p, 1.0 - p)
                w = jnp.where(is_t, 0.25, 0.75)
                fl = -w * jnp.square(1.0 - pt) * jnp.log(pt + 1e-12)
                if lw == 128:
                    facc = facc + fl * maskc
                else:
                    cls_sum += jnp.sum(fl * maskc)

            xc = cnt_ref[i, :, :]
            bce = (jnp.maximum(xc, 0.0) - xc * tc
                   + jnp.log1p(jnp.exp(-jnp.abs(xc))))
            cnt_sum += jnp.sum(bce * mask)

            rp = reg_ref[i]
            rt = reg_t[i][:, off:off + hh]
            reg_sum += jnp.sum(jnp.abs(rp - rt) * mask)

            msk_sum += jnp.sum(mask)
            off += hh

    cls_sum += jnp.sum(facc)
    cls_out[0, 0] += cls_sum
    cnt_out[0, 0] += cnt_sum
    reg_out[0, 0] += reg_sum
    npos_out[0, 0] += msk_sum


@jax.jit
def kernel(cls_logits_0, cls_logits_1, cls_logits_2, cls_logits_3, cls_logits_4,
           cnt_logits_0, cnt_logits_1, cnt_logits_2, cnt_logits_3, cnt_logits_4,
           reg_preds_0, reg_preds_1, reg_preds_2, reg_preds_3, reg_preds_4,
           cls_targets, cnt_targets, reg_targets):
    cls = [c.reshape(_B, _CLASS_NUM, hh)
           for c, hh in zip((cls_logits_0, cls_logits_1, cls_logits_2,
                             cls_logits_3, cls_logits_4), _HH)]
    cnt = [c.reshape(_B, 1, hh)
           for c, hh in zip((cnt_logits_0, cnt_logits_1, cnt_logits_2,
                             cnt_logits_3, cnt_logits_4), _HH)]
    reg = [r.reshape(_B, 4, hh)
           for r, hh in zip((reg_preds_0, reg_preds_1, reg_preds_2,
                             reg_preds_3, reg_preds_4), _HH)]
    cls_t = cls_targets.reshape(_B, 1, _N)
    cnt_t = cnt_targets.reshape(_B, 1, _N)
    reg_t = jnp.transpose(reg_targets, (0, 2, 1))

    in_specs = (
        [pl.BlockSpec((_BB, _CLASS_NUM, hh), lambda b: (b, 0, 0)) for hh in _HH]
        + [pl.BlockSpec((_BB, 1, hh), lambda b: (b, 0, 0)) for hh in _HH]
        + [pl.BlockSpec((_BB, 4, hh), lambda b: (b, 0, 0)) for hh in _HH]
        + [pl.BlockSpec((_BB, 1, _N), lambda b: (b, 0, 0)),
           pl.BlockSpec((_BB, 1, _N), lambda b: (b, 0, 0)),
           pl.BlockSpec((_BB, 4, _N), lambda b: (b, 0, 0))]
    )
    out_specs = [pl.BlockSpec(memory_space=pltpu.SMEM)] * 4
    out_shape = [jax.ShapeDtypeStruct((1, 1), jnp.float32)] * 4

    cls_s, cnt_s, reg_s, npos = pl.pallas_call(
        _loss_body,
        grid=(_STEPS,),
        in_specs=in_specs,
        out_specs=out_specs,
        out_shape=out_shape,
    )(*cls, *cnt, *reg, cls_t, cnt_t, reg_t)

    num_pos = jnp.maximum(npos[0, 0], 1.0)
    cls_loss = cls_s[0, 0] / num_pos
    cnt_loss = cnt_s[0, 0] / num_pos
    reg_loss = reg_s[0, 0] / num_pos
    return (cls_loss, cnt_loss, reg_loss, cls_loss + cnt_loss + reg_loss)
